# baseline (speedup 1.0000x reference)
import jax
import jax.numpy as jnp
from jax.experimental import pallas as pl
from jax.experimental.pallas import tpu as pltpu

_ROWS = 16384
_COLS = 1000
_BLK = 256
_NBLK = _ROWS // _BLK
_K = _ROWS // 2


def _ohem_kernel(tgt_ref, x_ref, out_ref, loss_scr):
    i = pl.program_id(0)
    x = x_ref[...]
    tgt = tgt_ref[0, 0, :]
    m = jnp.max(x, axis=1)
    s = jnp.sum(jnp.exp(x - m[:, None]), axis=1)
    logz = jnp.log(s) + m
    cols = jax.lax.broadcasted_iota(jnp.int32, (_BLK, _COLS), 1)
    picked = jnp.sum(jnp.where(cols == tgt[:, None], x, 0.0), axis=1)
    loss_scr[i, :] = logz - picked

    @pl.when(i == _NBLK - 1)
    def _select():
        loss = loss_scr[...]
        ib = jax.lax.bitcast_convert_type(loss, jnp.int32)
        key = jnp.where(ib >= 0, ib, ib ^ jnp.int32(0x7FFFFFFF))
        key2 = key ^ jnp.int32(-2147483648)

        def body(t, carry):
            pmask, pval, kp = carry
            bit = jnp.left_shift(jnp.int32(1), 31 - t)
            m2 = pmask | bit
            want = pval | bit
            ones = jnp.sum(((key2 & m2) == want).astype(jnp.int32))
            take = ones >= kp
            pval = jnp.where(take, want, pval)
            kp = jnp.where(take, kp, kp - ones)
            return (m2, pval, kp)

        _, pval, _ = jax.lax.fori_loop(
            0, 32, body, (jnp.int32(0), jnp.int32(0), jnp.int32(_K)))
        t_key = pval ^ jnp.int32(-2147483648)
        mask_gt = key > t_key
        cnt_gt = jnp.sum(mask_gt.astype(jnp.int32))
        sum_gt = jnp.sum(jnp.where(mask_gt, loss, 0.0))
        t_bits = jnp.where(t_key >= 0, t_key, t_key ^ jnp.int32(0x7FFFFFFF))
        t_val = jax.lax.bitcast_convert_type(t_bits, jnp.float32)
        ans = (sum_gt + (_K - cnt_gt).astype(jnp.float32) * t_val) / _K
        out_ref[...] = jnp.broadcast_to(ans, (1, 1))


def kernel(input, target):
    tgt3 = target.astype(jnp.int32).reshape(_NBLK, 1, _BLK)
    out = pl.pallas_call(
        _ohem_kernel,
        grid=(_NBLK,),
        in_specs=[
            pl.BlockSpec((1, 1, _BLK), lambda i: (i, 0, 0)),
            pl.BlockSpec((_BLK, _COLS), lambda i: (i, 0)),
        ],
        out_specs=pl.BlockSpec((1, 1), lambda i: (0, 0)),
        out_shape=jax.ShapeDtypeStruct((1, 1), jnp.float32),
        scratch_shapes=[pltpu.VMEM((_NBLK, _BLK), jnp.float32)],
    )(tgt3, input)
    return out[0, 0]


# speedup vs baseline: 1.1664x; 1.1664x over previous
import jax
import jax.numpy as jnp
from jax.experimental import pallas as pl
from jax.experimental.pallas import tpu as pltpu

_ROWS = 16384
_COLS = 1000
_BLK = 512
_NBLK = _ROWS // _BLK
_K = _ROWS // 2


def _ohem_kernel(tgt_ref, x_ref, out_ref, loss_scr):
    i = pl.program_id(0)
    x = x_ref[...]
    tgt = tgt_ref[0, 0, :]
    m = jnp.max(x, axis=1)
    s = jnp.sum(jnp.exp(x - m[:, None]), axis=1)
    logz = jnp.log(s) + m
    cols = jax.lax.broadcasted_iota(jnp.int32, (_BLK, _COLS), 1)
    picked = jnp.sum(jnp.where(cols == tgt[:, None], x, 0.0), axis=1)
    loss_scr[i, :] = logz - picked

    @pl.when(i == _NBLK - 1)
    def _select():
        loss = loss_scr[...]
        ib = jax.lax.bitcast_convert_type(loss, jnp.int32)
        key = jnp.where(ib >= 0, ib, ib ^ jnp.int32(0x7FFFFFFF))
        key2 = key ^ jnp.int32(-2147483648)

        def body(t, carry):
            pmask, pval, kp = carry
            bit = jnp.left_shift(jnp.int32(1), 31 - t)
            m2 = pmask | bit
            want = pval | bit
            ones = jnp.sum(((key2 & m2) == want).astype(jnp.int32))
            take = ones >= kp
            pval = jnp.where(take, want, pval)
            kp = jnp.where(take, kp, kp - ones)
            return (m2, pval, kp)

        _, pval, _ = jax.lax.fori_loop(
            0, 32, body, (jnp.int32(0), jnp.int32(0), jnp.int32(_K)))
        t_key = pval ^ jnp.int32(-2147483648)
        mask_gt = key > t_key
        cnt_gt = jnp.sum(mask_gt.astype(jnp.int32))
        sum_gt = jnp.sum(jnp.where(mask_gt, loss, 0.0))
        t_bits = jnp.where(t_key >= 0, t_key, t_key ^ jnp.int32(0x7FFFFFFF))
        t_val = jax.lax.bitcast_convert_type(t_bits, jnp.float32)
        ans = (sum_gt + (_K - cnt_gt).astype(jnp.float32) * t_val) / _K
        out_ref[...] = jnp.broadcast_to(ans, (1, 1))


def kernel(input, target):
    tgt3 = target.astype(jnp.int32).reshape(_NBLK, 1, _BLK)
    out = pl.pallas_call(
        _ohem_kernel,
        grid=(_NBLK,),
        in_specs=[
            pl.BlockSpec((1, 1, _BLK), lambda i: (i, 0, 0)),
            pl.BlockSpec((_BLK, _COLS), lambda i: (i, 0)),
        ],
        out_specs=pl.BlockSpec((1, 1), lambda i: (0, 0)),
        out_shape=jax.ShapeDtypeStruct((1, 1), jnp.float32),
        scratch_shapes=[pltpu.VMEM((_NBLK, _BLK), jnp.float32)],
    )(tgt3, input)
    return out[0, 0]


# speedup vs baseline: 1.2718x; 1.0903x over previous
import jax
import jax.numpy as jnp
from jax.experimental import pallas as pl
from jax.experimental.pallas import tpu as pltpu

_ROWS = 16384
_COLS = 1000
_BLK = 1024
_NBLK = _ROWS // _BLK
_K = _ROWS // 2


def _ohem_kernel(tgt_ref, x_ref, out_ref, loss_scr):
    i = pl.program_id(0)
    x = x_ref[...]
    tgt = tgt_ref[0, 0, :]
    m = jnp.max(x, axis=1)
    s = jnp.sum(jnp.exp(x - m[:, None]), axis=1)
    logz = jnp.log(s) + m
    cols = jax.lax.broadcasted_iota(jnp.int32, (_BLK, _COLS), 1)
    picked = jnp.sum(jnp.where(cols == tgt[:, None], x, 0.0), axis=1)
    loss_scr[i, :] = logz - picked

    @pl.when(i == _NBLK - 1)
    def _select():
        loss = loss_scr[...]
        ib = jax.lax.bitcast_convert_type(loss, jnp.int32)
        key = jnp.where(ib >= 0, ib, ib ^ jnp.int32(0x7FFFFFFF))
        key2 = key ^ jnp.int32(-2147483648)

        def body(t, carry):
            pmask, pval, kp = carry
            bit = jnp.left_shift(jnp.int32(1), 31 - t)
            m2 = pmask | bit
            want = pval | bit
            ones = jnp.sum(((key2 & m2) == want).astype(jnp.int32))
            take = ones >= kp
            pval = jnp.where(take, want, pval)
            kp = jnp.where(take, kp, kp - ones)
            return (m2, pval, kp)

        _, pval, _ = jax.lax.fori_loop(
            0, 32, body, (jnp.int32(0), jnp.int32(0), jnp.int32(_K)))
        t_key = pval ^ jnp.int32(-2147483648)
        mask_gt = key > t_key
        cnt_gt = jnp.sum(mask_gt.astype(jnp.int32))
        sum_gt = jnp.sum(jnp.where(mask_gt, loss, 0.0))
        t_bits = jnp.where(t_key >= 0, t_key, t_key ^ jnp.int32(0x7FFFFFFF))
        t_val = jax.lax.bitcast_convert_type(t_bits, jnp.float32)
        ans = (sum_gt + (_K - cnt_gt).astype(jnp.float32) * t_val) / _K
        out_ref[...] = jnp.broadcast_to(ans, (1, 1))


def kernel(input, target):
    tgt3 = target.astype(jnp.int32).reshape(_NBLK, 1, _BLK)
    out = pl.pallas_call(
        _ohem_kernel,
        grid=(_NBLK,),
        in_specs=[
            pl.BlockSpec((1, 1, _BLK), lambda i: (i, 0, 0)),
            pl.BlockSpec((_BLK, _COLS), lambda i: (i, 0)),
        ],
        out_specs=pl.BlockSpec((1, 1), lambda i: (0, 0)),
        out_shape=jax.ShapeDtypeStruct((1, 1), jnp.float32),
        scratch_shapes=[pltpu.VMEM((_NBLK, _BLK), jnp.float32)],
    )(tgt3, input)
    return out[0, 0]
